# baseline (speedup 1.0000x reference)
import functools

import jax
import jax.numpy as jnp
from jax import lax
from jax.experimental import pallas as pl
from jax.experimental.pallas import tpu as pltpu
from jax.experimental.pallas import tpu_sc as plsc

_VOCAB = 100000
_T = 200
_E = 64
_B = 4096

_NC = 2
_NS = 16
_NW = _NC * _NS
_BLK = _B // _NW
_NBUF = 4
_TB = _T // 8


def _sc_embed(vals2, tok_table, pos_table):
  mesh = plsc.VectorSubcoreMesh(core_axis_name="c", subcore_axis_name="s")

  @functools.partial(
      pl.kernel,
      mesh=mesh,
      compiler_params=pltpu.CompilerParams(
          use_tc_tiling_on_sc=False, needs_layout_passes=False),
      out_type=jax.ShapeDtypeStruct((_T, 8, _NW * 8, _BLK), jnp.float32),
      scratch_types=(
          [pltpu.VMEM((_TB, 1024), jnp.int32),
           pltpu.VMEM((_T, _E), jnp.float32)]
          + [pltpu.VMEM((_BLK, _E), jnp.float32) for _ in range(_NBUF)]
          + [pltpu.VMEM((8, 8, _BLK), jnp.float32) for _ in range(_NBUF)]
          + [pltpu.SemaphoreType.DMA for _ in range(2 * _NBUF)]
      ),
  )
  def k(vals_hbm, tok_hbm, pos_hbm, out_hbm, vidx, pos_v, *rest):
    gbufs = rest[:_NBUF]
    tbufs = rest[_NBUF:2 * _NBUF]
    gsems = rest[2 * _NBUF:3 * _NBUF]
    ssems = rest[3 * _NBUF:]

    wid = lax.axis_index("s") * _NC + lax.axis_index("c")
    pltpu.sync_copy(pos_hbm, pos_v)
    pltpu.sync_copy(vals_hbm.at[:, pl.ds(wid * 1024, 1024)], vidx)

    iota = lax.iota(jnp.int32, 16)
    ridx = [iota + (c * 16) for c in range(8)]

    def idx_slice(t):
      return vidx.at[t // 8, pl.ds((t % 8) * 128, 128)]

    def fire_gather(m, t):
      pltpu.async_copy(tok_hbm.at[idx_slice(t)], gbufs[m], gsems[m])

    def wait_gather(m, t):
      pltpu.make_async_copy(tok_hbm.at[idx_slice(t)], gbufs[m], gsems[m]).wait()

    def out_slice(t):
      return out_hbm.at[t, :, pl.ds(wid * 8, 8)]

    def fire_store(m, t):
      pltpu.async_copy(tbufs[m], out_slice(t), ssems[m])

    def wait_store(m, t):
      pltpu.make_async_copy(tbufs[m], out_slice(t), ssems[m]).wait()

    def transpose_add(m, tsplat):
      def ev_body(e_vec, eb_base):
        for dd in range(0, 16, 2):
          ev0 = eb_base + ((iota + dd) & 15) if dd else eb_base + iota
          ev1 = eb_base + ((iota + dd + 1) & 15)
          posr0 = plsc.load_gather(pos_v, [tsplat, ev0])
          posr1 = plsc.load_gather(pos_v, [tsplat, ev1])
          vs0 = [plsc.load_gather(gbufs[m], [ridx[c], ev0]) for c in range(8)]
          vs1 = [plsc.load_gather(gbufs[m], [ridx[c], ev1]) for c in range(8)]
          eh0, el0 = ev0 >> 3, ev0 & 7
          eh1, el1 = ev1 >> 3, ev1 & 7
          for c in range(8):
            plsc.store_scatter(tbufs[m], [eh0, el0, ridx[c]], vs0[c] + posr0)
            plsc.store_scatter(tbufs[m], [eh1, el1, ridx[c]], vs1[c] + posr1)
        return eb_base + 16
      lax.fori_loop(0, 4, ev_body, jnp.zeros((16,), jnp.int32))

    fire_gather(0, 0)
    fire_gather(1, 1)

    def step(j, tj):
      for kk in range(_NBUF):
        t = j * _NBUF + kk
        nxt2 = (kk + 2) % _NBUF

        if kk < 2:
          fire_gather(nxt2, t + 2)
        else:
          @pl.when(j < _T // _NBUF - 1)
          def _():
            fire_gather(nxt2, t + 2)

        wait_gather(kk, t)
        @pl.when(j > 0)
        def _():
          wait_store(kk, t - _NBUF)
        transpose_add(kk, tj + kk)
        fire_store(kk, t)
      return tj + _NBUF

    lax.fori_loop(0, _T // _NBUF, step, jnp.zeros((16,), jnp.int32))

    for kk in range(_NBUF):
      wait_store(kk, _T - _NBUF + kk)

  return k(vals2, tok_table, pos_table)


def kernel(values, tok_table, pos_table):
  v = values.astype(jnp.int32)
  vals2 = (v.T.reshape(_TB, 8, _NW, _BLK)
           .transpose(0, 2, 1, 3).reshape(_TB, _NW * 1024))
  out4 = _sc_embed(vals2, tok_table, pos_table)
  return (out4.reshape(_T, 8, _NW, 8, _BLK)
          .transpose(2, 4, 0, 1, 3).reshape(_B, _T, _E))


# speedup vs baseline: 1.3879x; 1.3879x over previous
import functools

import jax
import jax.numpy as jnp
from jax import lax
from jax.experimental import pallas as pl
from jax.experimental.pallas import tpu as pltpu
from jax.experimental.pallas import tpu_sc as plsc

_VOCAB = 100000
_T = 200
_E = 64
_B = 4096

_NC = 2
_NS = 16
_NW = _NC * _NS
_BLK = _B // _NW
_NBUF = 4
_TB = _T // 8


def _sc_embed(vals2, tok_table, pos_table):
  mesh = plsc.VectorSubcoreMesh(core_axis_name="c", subcore_axis_name="s")

  @functools.partial(
      pl.kernel,
      mesh=mesh,
      compiler_params=pltpu.CompilerParams(
          use_tc_tiling_on_sc=False, needs_layout_passes=False),
      out_type=jax.ShapeDtypeStruct((_B * _T * _E // 128, 128), jnp.float32),
      scratch_types=(
          [pltpu.VMEM((_TB, 1024), jnp.int32),
           pltpu.VMEM((_T, _E), jnp.float32)]
          + [pltpu.VMEM((_BLK, _E), jnp.float32) for _ in range(_NBUF)]
          + [pltpu.VMEM((_E, _BLK), jnp.float32) for _ in range(_NBUF)]
          + [pltpu.SemaphoreType.DMA for _ in range(2 * _NBUF)]
      ),
  )
  def k(vals_hbm, tok_hbm, pos_hbm, out_hbm, vidx, pos_v, *rest):
    gbufs = rest[:_NBUF]
    tbufs = rest[_NBUF:2 * _NBUF]
    gsems = rest[2 * _NBUF:3 * _NBUF]
    ssems = rest[3 * _NBUF:]

    wid = lax.axis_index("s") * _NC + lax.axis_index("c")
    pltpu.sync_copy(pos_hbm, pos_v)
    pltpu.sync_copy(vals_hbm.at[:, pl.ds(wid * 1024, 1024)], vidx)

    iota = lax.iota(jnp.int32, 16)
    ridx = [iota + (c * 16) for c in range(8)]

    def idx_slice(t):
      return vidx.at[t // 8, pl.ds((t % 8) * 128, 128)]

    def fire_gather(m, t):
      pltpu.async_copy(tok_hbm.at[idx_slice(t)], gbufs[m], gsems[m])

    def wait_gather(m, t):
      pltpu.make_async_copy(tok_hbm.at[idx_slice(t)], gbufs[m], gsems[m]).wait()

    def fire_store(m, t):
      for e_hi in range(8):
        pltpu.async_copy(
            tbufs[m].at[pl.ds(e_hi * 8, 8)],
            out_hbm.at[pl.ds(((t * 8 + e_hi) * _NW + wid) * 8, 8)],
            ssems[m])

    def wait_store(m, t):
      del t
      pltpu.make_async_copy(tbufs[m], out_hbm.at[pl.ds(0, _E)], ssems[m]).wait()

    def transpose_add(m, tsplat):
      def ev_body(e_vec, eb_base):
        for dd in range(0, 16, 2):
          ev0 = eb_base + ((iota + dd) & 15) if dd else eb_base + iota
          ev1 = eb_base + ((iota + dd + 1) & 15)
          posr0 = plsc.load_gather(pos_v, [tsplat, ev0])
          posr1 = plsc.load_gather(pos_v, [tsplat, ev1])
          vs0 = [plsc.load_gather(gbufs[m], [ridx[c], ev0]) for c in range(8)]
          vs1 = [plsc.load_gather(gbufs[m], [ridx[c], ev1]) for c in range(8)]
          for c in range(8):
            plsc.store_scatter(tbufs[m], [ev0, ridx[c]], vs0[c] + posr0)
            plsc.store_scatter(tbufs[m], [ev1, ridx[c]], vs1[c] + posr1)
        return eb_base + 16
      lax.fori_loop(0, 4, ev_body, jnp.zeros((16,), jnp.int32))

    fire_gather(0, 0)
    fire_gather(1, 1)

    def step(j, tj):
      for kk in range(_NBUF):
        t = j * _NBUF + kk
        nxt2 = (kk + 2) % _NBUF

        if kk < 2:
          fire_gather(nxt2, t + 2)
        else:
          @pl.when(j < _T // _NBUF - 1)
          def _():
            fire_gather(nxt2, t + 2)

        wait_gather(kk, t)
        @pl.when(j > 0)
        def _():
          wait_store(kk, t - _NBUF)
        transpose_add(kk, tj + kk)
        fire_store(kk, t)
      return tj + _NBUF

    lax.fori_loop(0, _T // _NBUF, step, jnp.zeros((16,), jnp.int32))

    for kk in range(_NBUF):
      wait_store(kk, _T - _NBUF + kk)

  return k(vals2, tok_table, pos_table)


def kernel(values, tok_table, pos_table):
  v = values.astype(jnp.int32)
  vals2 = (v.T.reshape(_TB, 8, _NW, _BLK)
           .transpose(0, 2, 1, 3).reshape(_TB, _NW * 1024))
  out2 = _sc_embed(vals2, tok_table, pos_table)
  return (out2.reshape(_T, 8, _NW, 8, _BLK)
          .transpose(2, 4, 0, 1, 3).reshape(_B, _T, _E))
